# Initial kernel scaffold; baseline (speedup 1.0000x reference)
import jax
import jax.numpy as jnp
from jax.experimental import pallas as pl


def kernel(x, adj_matrix, type_w1, type_b1, type_w2, type_b2, adj_w1_d1, adj_b1_d1, adj_w2_d1, adj_b2_d1, adj_w1_d2, adj_b1_d2, adj_w2_d2, adj_b2_d2, main_w1, main_b1, main_w2, main_b2, w1_blk, b1_cat, w2_blk, b2_cat, wm1_x, bm1_x, wm2_x, bm2_x):
    raise NotImplementedError("write your pallas kernel here")


# speedup vs baseline: 2.1052x; 2.1052x over previous
import functools

import jax
import jax.numpy as jnp
from jax.experimental import pallas as pl
from jax.experimental.pallas import tpu as pltpu

_BF16 = jnp.bfloat16
_SQRT1_2 = 0.7071067811865476


def _erf_approx(x):
    a1, a2, a3, a4, a5 = 0.254829592, -0.284496736, 1.421413741, -1.453152027, 1.061405429
    p = 0.3275911
    sgn = jnp.where(x >= 0, 1.0, -1.0).astype(x.dtype)
    ax = jnp.abs(x)
    t = 1.0 / (1.0 + p * ax)
    poly = ((((a5 * t + a4) * t + a3) * t + a2) * t + a1) * t
    return sgn * (1.0 - poly * jnp.exp(-ax * ax))


def _gelu(x):
    return 0.5 * x * (1.0 + _erf_approx(x * _SQRT1_2))


def _reshape_rm(x, rows, cols):
    r0 = x.shape[0]
    flat = jnp.concatenate([x[i:i + 1, :] for i in range(r0)], axis=1)
    return jnp.concatenate([flat[:, i * cols:(i + 1) * cols] for i in range(rows)],
                           axis=0)


def _main_body(x_ref, w1_ref, b1c_ref, w2b_ref, b2c_ref,
               wm1_ref, bm1_ref, wm2_ref, bm2_ref, o_ref):
    bb, T, n, s = x_ref.shape
    rows = bb * n
    hs = []
    for t in range(T):
        xt = x_ref[:, t, :, :].reshape(rows, s).astype(_BF16)
        hs.append(jnp.dot(xt, w1_ref[t].astype(_BF16),
                          preferred_element_type=jnp.float32))
    h = jnp.concatenate(hs, axis=1) + b1c_ref[...]
    h = _gelu(h).astype(_BF16)
    e = jnp.dot(h, w2b_ref[...].astype(_BF16),
                preferred_element_type=jnp.float32) + b2c_ref[...]
    h2 = _gelu(jnp.dot(e.astype(_BF16), wm1_ref[...].astype(_BF16),
                       preferred_element_type=jnp.float32) + bm1_ref[...])
    o_ref[...] = jnp.dot(h2.astype(_BF16), wm2_ref[...].astype(_BF16),
                         preferred_element_type=jnp.float32) + bm2_ref[...]


def _main_forward(x, w1, b1c, w2b, b2c, wm1, bm1, wm2, bm2, *, block_b):
    B, T, N, S = x.shape
    out_cols = wm2.shape[1]
    grid = (B // block_b,)
    in_specs = [pl.BlockSpec((block_b, T, N, S), lambda i: (i, 0, 0, 0))]
    for w in (w1, b1c, w2b, b2c, wm1, bm1, wm2, bm2):
        nd = len(w.shape)
        in_specs.append(pl.BlockSpec(w.shape, lambda i, _nd=nd: (0,) * _nd))
    return pl.pallas_call(
        _main_body,
        out_shape=jax.ShapeDtypeStruct((B * N, out_cols), jnp.float32),
        grid=grid,
        in_specs=in_specs,
        out_specs=pl.BlockSpec((block_b * N, out_cols), lambda i: (i, 0)),
        compiler_params=pltpu.CompilerParams(dimension_semantics=("parallel",)),
    )(x, w1, b1c, w2b, b2c, wm1, bm1, wm2, bm2)


def _adj_body(a_ref, w11_ref, b11_ref, w21_ref, b21_ref,
              w12_ref, b12_ref, w22_ref, b22_ref, o_ref, *, k):
    n_in = a_ref.shape[0]
    h1 = w11_ref.shape[1]
    od = w22_ref.shape[1]

    h = _gelu(jnp.dot(a_ref[...], w11_ref[...],
                      preferred_element_type=jnp.float32) + b11_ref[...])
    h = _reshape_rm(h, h1, n_in)
    h = _gelu(jnp.dot(h, w12_ref[...],
                      preferred_element_type=jnp.float32) + b12_ref[...])
    h = jnp.dot(h, w22_ref[...], preferred_element_type=jnp.float32) + b22_ref[...]
    h = _reshape_rm(h, od, h1)
    sc = jnp.dot(h, w21_ref[...], preferred_element_type=jnp.float32) + b21_ref[...]

    ex = jnp.exp(sc - jnp.max(sc, axis=0, keepdims=True))
    sm = ex / jnp.sum(ex, axis=0, keepdims=True)

    rid = jax.lax.broadcasted_iota(jnp.int32, sm.shape, 0)
    work = sm
    thr = jnp.full((1, sm.shape[1]), -jnp.inf, jnp.float32)
    for _ in range(k):
        thr = jnp.max(work, axis=0, keepdims=True)
        first = jnp.min(jnp.where(work == thr, rid, sm.shape[0]),
                        axis=0, keepdims=True)
        work = jnp.where(rid == first, -jnp.inf, work)
    o_ref[...] = jnp.where(sm < thr, 0.0, sm)


def kernel(x, adj_matrix, type_w1, type_b1, type_w2, type_b2,
           adj_w1_d1, adj_b1_d1, adj_w2_d1, adj_b2_d1,
           adj_w1_d2, adj_b1_d2, adj_w2_d2, adj_b2_d2,
           main_w1, main_b1, main_w2, main_b2,
           w1_blk, b1_cat, w2_blk, b2_cat, wm1_x, bm1_x, wm2_x, bm2_x):
    B, T, N, S = x.shape
    Ot = type_w2.shape[2]
    Om = main_w2.shape[1]

    block_b = 8 if B % 8 == 0 else (2 if B % 2 == 0 else 1)
    out = _main_forward(x, type_w1, b1_cat, w2_blk, b2_cat,
                        wm1_x, bm1_x, wm2_x, bm2_x, block_b=block_b)
    y = out.reshape(B, N, Ot, Om).transpose(0, 3, 1, 2)

    o_adj = adj_w2_d1.shape[1]
    adj = pl.pallas_call(
        functools.partial(_adj_body, k=4),
        out_shape=jax.ShapeDtypeStruct((o_adj, o_adj), jnp.float32),
    )(adj_matrix,
      adj_w1_d1, adj_b1_d1, adj_w2_d1, adj_b2_d1,
      adj_w1_d2, adj_b1_d2, adj_w2_d2, adj_b2_d2)

    return y, adj
